# baseline (speedup 1.0000x reference)
import functools

import jax
import jax.numpy as jnp
from jax import lax
from jax.experimental import pallas as pl
from jax.experimental.pallas import tpu as pltpu
from jax.experimental.pallas import tpu_sc as plsc

N = 16384
D = 64
NUM_TABLES = 4
L = 16
WIN = 128
TAIL_BASE = 99968
NWIN = 782
SLOT_W = 98
CAP = N + WIN
SROWS = 160
TRASH = 152
PAD_ROWS = 16


@functools.cache
def _build(nc: int):
    mesh = plsc.VectorSubcoreMesh(core_axis_name="c", subcore_axis_name="s")

    @functools.partial(
        pl.kernel,
        out_type=jax.ShapeDtypeStruct((N + PAD_ROWS, 2 * D), jnp.float32),
        mesh=mesh,
        compiler_params=pltpu.CompilerParams(use_tc_tiling_on_sc=True,
                                             needs_layout_passes=False),
        scratch_types=[
            pltpu.VMEM((N,), jnp.int32),
            pltpu.VMEM((N,), jnp.int32),
            pltpu.VMEM((CAP,), jnp.int32),
            pltpu.VMEM((CAP,), jnp.int32),
            pltpu.VMEM((D, WIN), jnp.float32),
            pltpu.VMEM((D, WIN), jnp.float32),
            pltpu.VMEM((SROWS, 2 * D), jnp.float32),
            pltpu.VMEM((384,), jnp.int32),
            pltpu.VMEM((8, WIN), jnp.int32),
            pltpu.SemaphoreType.DMA,
            pltpu.SemaphoreType.DMA,
            pltpu.SemaphoreType.DMA,
        ],
    )
    def hetero_gather(x_hbm, t_hbm, w0, w1, w2, w3, tails_hbm, out_hbm,
                      x_v, t_v, xr_v, pos_v, wva, wvb, rows_v, posf_v,
                      pos2_v, sema, semb, semo):
        wid = lax.axis_index("s") * nc + lax.axis_index("c")
        my_t = wid % NUM_TABLES
        slot = wid // NUM_TABLES
        wlo = slot * SLOT_W
        whi = jnp.minimum(wlo + SLOT_W, NWIN)

        cp_x = pltpu.async_copy(x_hbm, x_v, sema)
        cp_t = pltpu.async_copy(t_hbm, t_v, semb)
        cp_x.wait()
        cp_t.wait()

        lanes = lax.iota(jnp.int32, L)
        tables = [w0, w1, w2, w3]

        def scan_body(i, cnt):
            s = pl.ds(i * L, L)
            xv = x_v[s]
            tv = t_v[s]
            xr = jnp.where(xv < TAIL_BASE, xv,
                           TAIL_BASE + my_t * (100000 - TAIL_BASE)
                           + xv - TAIL_BASE)
            wk = xr >> 7
            m = (tv == my_t) & (wk >= wlo) & (wk < whi)
            plsc.store_compressed(xr_v.at[pl.ds(cnt, L)], xr, mask=m)
            plsc.store_compressed(pos_v.at[pl.ds(cnt, L)], i * L + lanes,
                                  mask=m)
            return cnt + jnp.sum(m.astype(jnp.int32))
        cnt = lax.fori_loop(0, N // L, scan_body, jnp.int32(0), unroll=2)
        xr_v[pl.ds(cnt, L)] = jnp.full((L,), 1 << 20, jnp.int32)
        ng = (cnt + L - 1) // L

        def start(w, buf, sem):
            @pl.when(w < NWIN - 1)
            def _():
                for tt in range(NUM_TABLES):
                    @pl.when(my_t == tt)
                    def _(tt=tt):
                        pltpu.async_copy(
                            tables[tt].at[:, pl.ds(w * WIN, WIN)], buf, sem)

            @pl.when(w == NWIN - 1)
            def _():
                pltpu.async_copy(tails_hbm, buf, sem)

        def wait(buf, sem):
            pltpu.make_async_copy(tails_hbm, buf, sem).wait()

        def extract(w, buf, flcnt):
            def group(g, fl):
                xrv = xr_v[pl.ds(g * L, L)]
                m = (xrv >> 7) == w
                c16 = jnp.sum(m.astype(jnp.int32))

                @pl.when(c16 > 0)
                def _():
                    rcv = xrv & (WIN - 1)
                    posv = pos_v[pl.ds(g * L, L)]
                    cum = plsc.cumsum(m.astype(jnp.int32))
                    slotv = jnp.where(m, fl + cum - 1, TRASH)
                    plsc.store_scatter(posf_v, [slotv], posv)

                    def colb(c, _):
                        cv = jnp.full((L,), 0, jnp.int32) + c
                        v = plsc.load_gather(buf, [cv, rcv])
                        plsc.store_scatter(rows_v, [slotv, cv], v)
                        return 0
                    lax.fori_loop(0, D, colb, 0, unroll=4)

                fl2 = fl + c16

                @pl.when(fl2 >= WIN)
                def _():
                    for k in range(WIN // L):
                        s = pl.ds(k * L, L)
                        pos2_v[0, s] = posf_v[s]
                    pltpu.async_copy(rows_v.at[pl.ds(0, WIN)],
                                     out_hbm.at[pos2_v.at[0]], semo).wait()
                    ov = fl2 - WIN

                    def mv(j, _):
                        for k in range(2 * D // L):
                            s = pl.ds(k * L, L)
                            rows_v[j, s] = rows_v[WIN + j, s]
                        return 0
                    lax.fori_loop(0, ov, mv, 0)
                    posf_v[pl.ds(0, L)] = posf_v[pl.ds(WIN, L)]

                return jnp.where(fl2 >= WIN, fl2 - WIN, fl2)
            return lax.fori_loop(0, ng, group, flcnt)

        npair = (whi - wlo) // 2
        start(wlo, wva, sema)
        start(wlo + 1, wvb, semb)

        def pair(j, flcnt):
            w = wlo + 2 * j
            wait(wva, sema)
            flcnt = extract(w, wva, flcnt)

            @pl.when(j + 1 < npair)
            def _():
                start(w + 2, wva, sema)
            wait(wvb, semb)
            flcnt = extract(w + 1, wvb, flcnt)

            @pl.when(j + 1 < npair)
            def _():
                start(w + 3, wvb, semb)
            return flcnt
        flcnt = lax.fori_loop(0, npair, pair, jnp.int32(0))

        def padp(k, _):
            posf_v[pl.ds(flcnt + k * L, L)] = N + (lanes & (PAD_ROWS - 1))
            return 0
        lax.fori_loop(0, WIN // L, padp, 0)
        for k in range(WIN // L):
            s = pl.ds(k * L, L)
            pos2_v[0, s] = posf_v[s]
        pltpu.async_copy(rows_v.at[pl.ds(0, WIN)],
                         out_hbm.at[pos2_v.at[0]], semo).wait()

    return hetero_gather


def kernel(x, types, W0, W1, W2, W3):
    info = plsc.get_sparse_core_info()
    fn = _build(info.num_cores)
    wts = [W.T for W in (W0, W1, W2, W3)]
    tails = jnp.concatenate(
        [W[TAIL_BASE:] for W in (W0, W1, W2, W3)], axis=0).T
    out = fn(x.astype(jnp.int32), types.astype(jnp.int32), *wts, tails)
    return out[:N, :D]


# speedup vs baseline: 2.2128x; 2.2128x over previous
import functools

import jax
import jax.numpy as jnp
from jax import lax
from jax.experimental import pallas as pl
from jax.experimental.pallas import tpu as pltpu
from jax.experimental.pallas import tpu_sc as plsc

N = 16384
D = 64
NUM_TABLES = 4
L = 16
WIN = 128
TAIL_BASE = 99968
NWIN = 782
SLOT_W = 98
CAP = N + WIN
SROWS = 160
TRASH = 152
PAD_ROWS = 16
NB = 16


@functools.cache
def _build(nc: int):
    mesh = plsc.VectorSubcoreMesh(core_axis_name="c", subcore_axis_name="s")

    @functools.partial(
        pl.kernel,
        out_type=jax.ShapeDtypeStruct((N + PAD_ROWS, 2 * D), jnp.float32),
        mesh=mesh,
        compiler_params=pltpu.CompilerParams(use_tc_tiling_on_sc=True,
                                             needs_layout_passes=False),
        scratch_types=[
            pltpu.VMEM((N,), jnp.int32),
            pltpu.VMEM((N,), jnp.int32),
            pltpu.VMEM((CAP,), jnp.int32),
            pltpu.VMEM((CAP,), jnp.int32),
            pltpu.VMEM((128,), jnp.int32),
            pltpu.VMEM((D, WIN), jnp.float32),
            pltpu.VMEM((D, WIN), jnp.float32),
            pltpu.VMEM((SROWS, 2 * D), jnp.float32),
            pltpu.VMEM((384,), jnp.int32),
            pltpu.VMEM((8, WIN), jnp.int32),
            pltpu.SemaphoreType.DMA,
            pltpu.SemaphoreType.DMA,
            pltpu.SemaphoreType.DMA,
        ],
    )
    def hetero_gather(x_hbm, t_hbm, w0, w1, w2, w3, tails_hbm, out_hbm,
                      x_v, t_v, xr_v, pos_v, off_v,
                      wva, wvb, rows_v, posf_v, pos2_v, sema, semb, semo):
        wid = lax.axis_index("s") * nc + lax.axis_index("c")
        my_t = wid % NUM_TABLES
        slot = wid // NUM_TABLES
        wlo = slot * SLOT_W
        whi = jnp.minimum(wlo + SLOT_W, NWIN)

        cp_x = pltpu.async_copy(x_hbm, x_v, sema)
        cp_t = pltpu.async_copy(t_hbm, t_v, semb)
        cp_x.wait()
        cp_t.wait()

        lanes = lax.iota(jnp.int32, L)
        tables = [w0, w1, w2, w3]

        def scan_body(i, cnt):
            s = pl.ds(i * L, L)
            xv = x_v[s]
            tv = t_v[s]
            xr = jnp.where(xv < TAIL_BASE, xv,
                           TAIL_BASE + my_t * (100000 - TAIL_BASE)
                           + xv - TAIL_BASE)
            wk = xr >> 7
            m = (tv == my_t) & (wk >= wlo) & (wk < whi)
            plsc.store_compressed(xr_v.at[pl.ds(cnt, L)], xr, mask=m)
            plsc.store_compressed(pos_v.at[pl.ds(cnt, L)], i * L + lanes,
                                  mask=m)
            return cnt + plsc.all_reduce_population_count(m)[0]
        cnt = lax.fori_loop(0, N // L, scan_body, jnp.int32(0), unroll=2)

        @pl.when(cnt < N)
        def _():
            xr_v[pl.ds(cnt, L)] = jnp.full((L,), 1 << 20, jnp.int32)
        ng = (cnt + L - 1) // L

        xr2_v, pos2b_v = x_v, t_v
        bcnt = [jnp.int32(0)] * NB
        def count_pass(g, carry):
            bv = (xr_v[pl.ds(g * L, L)] >> 7) & (NB - 1)
            out = []
            for b in range(NB):
                out.append(carry[b]
                           + plsc.all_reduce_population_count(bv == b)[0])
            return tuple(out)
        bcnt = lax.fori_loop(0, ng, count_pass, tuple(bcnt))
        boff = []
        acc = jnp.int32(0)
        for b in range(NB):
            boff.append(acc)
            acc = acc + bcnt[b]
        offv = jnp.zeros((L,), jnp.int32)
        limv = jnp.zeros((L,), jnp.int32)
        for b in range(NB):
            offv = jnp.where(lanes == b, boff[b], offv)
            limv = jnp.where(lanes == b, boff[b] + bcnt[b], limv)
        off_v[pl.ds(0, L)] = offv
        off_v[pl.ds(L, L)] = limv

        def place_pass(g, carry):
            s = pl.ds(g * L, L)
            xrv = xr_v[s]
            posv = pos_v[s]
            bv = (xrv >> 7) & (NB - 1)
            out = []
            for b in range(NB):
                m = bv == b
                plsc.store_compressed(xr2_v.at[pl.ds(carry[b], L)], xrv,
                                      mask=m)
                plsc.store_compressed(pos2b_v.at[pl.ds(carry[b], L)], posv,
                                      mask=m)
                out.append(carry[b]
                           + plsc.all_reduce_population_count(m)[0])
            return tuple(out)
        lax.fori_loop(0, ng, place_pass, tuple(boff))
        @pl.when(acc < N)
        def _():
            xr2_v[pl.ds(acc, L)] = jnp.full((L,), 1 << 20, jnp.int32)

        def start(w, buf, sem):
            @pl.when(w < NWIN - 1)
            def _():
                for tt in range(NUM_TABLES):
                    @pl.when(my_t == tt)
                    def _(tt=tt):
                        pltpu.async_copy(
                            tables[tt].at[:, pl.ds(w * WIN, WIN)], buf, sem)

            @pl.when(w == NWIN - 1)
            def _():
                pltpu.async_copy(tails_hbm, buf, sem)

        def wait(buf, sem):
            pltpu.make_async_copy(tails_hbm, buf, sem).wait()

        def extract(w, buf, flcnt):
            b = w & (NB - 1)
            blo = off_v[pl.ds(b, L)][0]
            bhi = off_v[pl.ds(L + b, L)][0]

            def group(g, fl):
                xrv = xr2_v[pl.ds(g * L, L)]
                m = (xrv >> 7) == w
                c16 = plsc.all_reduce_population_count(m)[0]

                @pl.when(c16 > 0)
                def _():
                    rcv = xrv & (WIN - 1)
                    posv = pos2b_v[pl.ds(g * L, L)]
                    cum = plsc.cumsum(m.astype(jnp.int32))
                    slotv = jnp.where(m, fl + cum - 1, TRASH)
                    plsc.store_scatter(posf_v, [slotv], posv)

                    def colb(c, _):
                        cv = jnp.full((L,), 0, jnp.int32) + c
                        v = plsc.load_gather(buf, [cv, rcv])
                        plsc.store_scatter(rows_v, [slotv, cv], v)
                        return 0
                    lax.fori_loop(0, D, colb, 0, unroll=4)

                fl2 = fl + c16

                @pl.when(fl2 >= WIN)
                def _():
                    for k in range(WIN // L):
                        s = pl.ds(k * L, L)
                        pos2_v[0, s] = posf_v[s]
                    pltpu.async_copy(rows_v.at[pl.ds(0, WIN)],
                                     out_hbm.at[pos2_v.at[0]], semo).wait()
                    ov = fl2 - WIN

                    def mv(j, _):
                        for k in range(2 * D // L):
                            s = pl.ds(k * L, L)
                            rows_v[j, s] = rows_v[WIN + j, s]
                        return 0
                    lax.fori_loop(0, ov, mv, 0)
                    posf_v[pl.ds(0, L)] = posf_v[pl.ds(WIN, L)]

                return jnp.where(fl2 >= WIN, fl2 - WIN, fl2)
            return lax.fori_loop(blo >> 4, (bhi + L - 1) >> 4, group, flcnt)

        npair = (whi - wlo) // 2
        start(wlo, wva, sema)
        start(wlo + 1, wvb, semb)

        def pair(j, flcnt):
            w = wlo + 2 * j
            wait(wva, sema)
            flcnt = extract(w, wva, flcnt)

            @pl.when(j + 1 < npair)
            def _():
                start(w + 2, wva, sema)
            wait(wvb, semb)
            flcnt = extract(w + 1, wvb, flcnt)

            @pl.when(j + 1 < npair)
            def _():
                start(w + 3, wvb, semb)
            return flcnt
        flcnt = lax.fori_loop(0, npair, pair, jnp.int32(0))

        def padp(k, _):
            posf_v[pl.ds(flcnt + k * L, L)] = N + (lanes & (PAD_ROWS - 1))
            return 0
        lax.fori_loop(0, WIN // L, padp, 0)
        for k in range(WIN // L):
            s = pl.ds(k * L, L)
            pos2_v[0, s] = posf_v[s]
        pltpu.async_copy(rows_v.at[pl.ds(0, WIN)],
                         out_hbm.at[pos2_v.at[0]], semo).wait()

    return hetero_gather


def kernel(x, types, W0, W1, W2, W3):
    info = plsc.get_sparse_core_info()
    fn = _build(info.num_cores)
    wts = [W.T for W in (W0, W1, W2, W3)]
    tails = jnp.concatenate(
        [W[TAIL_BASE:] for W in (W0, W1, W2, W3)], axis=0).T
    out = fn(x.astype(jnp.int32), types.astype(jnp.int32), *wts, tails)
    return out[:N, :D]
